# speedup vs baseline: 79.1567x; 79.1567x over previous
import jax
import jax.numpy as jnp
from jax import lax
from jax.experimental import pallas as pl
from jax.experimental.pallas import tpu as pltpu
from jax.experimental.pallas import tpu_sc as plsc

_N = 50000
_E = 1600000
_ND = 250000
_NC, _NS, _L = 2, 16, 16
_NW = _NC * _NS
_ZPAD = 250880
_CH = _ZPAD // _NS
_EPT = _E // _NW
_K = 10000
_NB = _EPT // _K
_NPAD = 51200
_CHN = _NPAD // _NS

_mesh = plsc.VectorSubcoreMesh(core_axis_name="c", subcore_axis_name="s")


def _seg_body(pre_hbm, post_hbm, w_hbm, z_hbm, out_hbm,
              idx_v, post_v, w_v, g_v, zstage, zblk, zsp, accsp, sem):
    cid = lax.axis_index("c")
    sid = lax.axis_index("s")
    wid = sid * _NC + cid

    pltpu.sync_copy(z_hbm.at[pl.ds(sid * _CH, _CH)], zstage)
    pltpu.sync_copy(zstage, zsp.at[pl.ds(sid * _CH, _CH)])

    def _zero(i, c):
        zblk[pl.ds(pl.multiple_of(i * _L, _L), _L)] = jnp.zeros((_L,), jnp.float32)
        return c
    lax.fori_loop(0, _CHN // _L, _zero, 0)
    pltpu.sync_copy(zblk, accsp.at[pl.ds(sid * _CHN, _CHN)])
    plsc.subcore_barrier()

    for b in range(_NB):
        base = wid * _EPT + b * _K
        pltpu.sync_copy(pre_hbm.at[pl.ds(base, _K)], idx_v)
        pltpu.sync_copy(w_hbm.at[pl.ds(base, _K)], w_v)
        pltpu.async_copy(zsp.at[idx_v], g_v, sem).wait()

        def _mul(i, c):
            s = pl.ds(pl.multiple_of(i * _L, _L), _L)
            g_v[s] = g_v[s] * w_v[s]
            return c
        lax.fori_loop(0, _K // _L, _mul, 0)

        pltpu.sync_copy(post_hbm.at[pl.ds(base, _K)], post_v)
        pltpu.sync_copy(g_v, accsp.at[post_v], add=True)

    plsc.subcore_barrier()
    pltpu.sync_copy(accsp.at[pl.ds(sid * _CHN, _CHN)], zblk)
    pltpu.sync_copy(zblk, out_hbm.at[pl.ds(cid * _NPAD + sid * _CHN, _CHN)])


_seg_sum = pl.kernel(
    _seg_body,
    out_type=jax.ShapeDtypeStruct((_NC * _NPAD,), jnp.float32),
    mesh=_mesh,
    scratch_types=[
        pltpu.VMEM((_K,), jnp.int32),
        pltpu.VMEM((_K,), jnp.int32),
        pltpu.VMEM((_K,), jnp.float32),
        pltpu.VMEM((_K,), jnp.float32),
        pltpu.VMEM((_CH,), jnp.float32),
        pltpu.VMEM((_CHN,), jnp.float32),
        pltpu.VMEM_SHARED((_ZPAD,), jnp.float32),
        pltpu.VMEM_SHARED((_NPAD,), jnp.float32),
        pltpu.SemaphoreType.DMA,
    ],
)


def _glif_body(p0_ref, p1_ref, v_ref, ext_ref, decay_ref, cf_ref,
               vth_ref, vreset_ref, el_ref, out_ref):
    rec = p0_ref[...] + p1_ref[...]
    new_v = decay_ref[...] * v_ref[...] + cf_ref[...] * (rec + ext_ref[...])
    v_scaled = (new_v - vth_ref[...]) / (vth_ref[...] - el_ref[...] + 1e-8)
    z = (v_scaled > 0.0).astype(jnp.float32)
    v_out = new_v * (1.0 - z) + vreset_ref[...] * z
    out_ref[0:1, :] = z
    out_ref[1:2, :] = v_out


def kernel(z_buf, v, ext_current, rec_weights, decay, current_factor,
           v_th, v_reset, e_l, rec_indices):
    pre = rec_indices[:, 1]
    post = rec_indices[:, 0]
    zflat = jnp.pad(z_buf.reshape(-1), (0, _ZPAD - _ND))
    partial = _seg_sum(pre, post, rec_weights, zflat)
    p0 = partial[:_N][None, :]
    p1 = partial[_NPAD:_NPAD + _N][None, :]
    out2 = pl.pallas_call(
        _glif_body,
        out_shape=jax.ShapeDtypeStruct((2, _N), jnp.float32),
    )(p0, p1, v, ext_current, decay[None, :], current_factor[None, :],
      v_th[None, :], v_reset[None, :], e_l[None, :])
    return out2.reshape(1, 2 * _N)
